# Initial kernel scaffold; baseline (speedup 1.0000x reference)
import jax
import jax.numpy as jnp
from jax.experimental import pallas as pl


def kernel(x, idx0, idx1, idx2, idx3, drow0, dcol0, dval0, drow1, dcol1, dval1, drow2, dcol2, dval2, drow3, dcol3, dval3, urow0, ucol0, uval0, urow1, ucol1, uval1, urow2, ucol2, uval2, urow3, ucol3, uval3, We0, be0, We1, be1, We2, be2, We3, be3, Wmu, bmu, Wdl, bdl, Wd1, bd1, Wd2, bd2, Wd3, bd3, Wd4, bd4, Wo, bo):
    raise NotImplementedError("write your pallas kernel here")


# speedup vs baseline: 4.2861x; 4.2861x over previous
import functools

import jax
import jax.numpy as jnp
from jax import lax
from jax.experimental import pallas as pl
from jax.experimental.pallas import tpu as pltpu
from jax.experimental.pallas import tpu_sc as plsc

NC = 2
NS = 16
NW = NC * NS


def _cdiv(a, b):
    return -(-a // b)


@functools.lru_cache(maxsize=None)
def _sc_gather_fn(N, D, M_pad, ch, iters, per_w):
    mesh = plsc.VectorSubcoreMesh(core_axis_name="c", subcore_axis_name="s")

    @functools.partial(
        pl.kernel,
        out_type=jax.ShapeDtypeStruct((M_pad, D), jnp.float32),
        mesh=mesh,
        scratch_types=[
            pltpu.VMEM((ch,), jnp.int32),
            pltpu.VMEM((ch, D), jnp.float32),
            pltpu.SemaphoreType.DMA,
        ],
        compiler_params=pltpu.CompilerParams(use_tc_tiling_on_sc=False),
    )
    def k(table_hbm, idx_hbm, out_hbm, idx_v, rows_v, sem):
        wid = lax.axis_index("s") * NC + lax.axis_index("c")
        base = wid * per_w

        def body(g, carry):
            o = base + g * ch
            pltpu.sync_copy(idx_hbm.at[pl.ds(o, ch)], idx_v)
            pltpu.async_copy(table_hbm.at[idx_v], rows_v, sem).wait()
            pltpu.sync_copy(rows_v, out_hbm.at[pl.ds(o, ch)])
            return carry

        lax.fori_loop(0, iters, body, 0)

    return k


def _sc_gather(table, idx):
    N, D = table.shape
    M = idx.shape[0]
    ch_cap = max(8, (400_000 // (4 * D)) // 8 * 8)
    per_w = _cdiv(M, NW)
    ch = min(ch_cap, _cdiv(per_w, 8) * 8)
    iters = _cdiv(per_w, ch)
    per_w = iters * ch
    M_pad = per_w * NW
    if M_pad != M:
        idx = jnp.pad(idx, (0, M_pad - M))
    out = _sc_gather_fn(N, D, M_pad, ch, iters, per_w)(table, idx)
    if M_pad != M:
        out = out[:M]
    return out


def _tc_linear(X, W, b, act):
    Nr, K = X.shape
    C = W.shape[1]
    BLK = min(1024, _cdiv(Nr, 8) * 8)
    grid = _cdiv(Nr, BLK)

    def body(x_ref, w_ref, b_ref, o_ref):
        acc = jnp.dot(x_ref[...], w_ref[...],
                      preferred_element_type=jnp.float32) + b_ref[...]
        if act == "elu":
            acc = jnp.where(acc > 0, acc, jnp.exp(jnp.minimum(acc, 0.0)) - 1.0)
        o_ref[...] = acc

    return pl.pallas_call(
        body,
        grid=(grid,),
        in_specs=[
            pl.BlockSpec((BLK, K), lambda i: (i, 0)),
            pl.BlockSpec((K, C), lambda i: (0, 0)),
            pl.BlockSpec((1, C), lambda i: (0, 0)),
        ],
        out_specs=pl.BlockSpec((BLK, C), lambda i: (i, 0)),
        out_shape=jax.ShapeDtypeStruct((Nr, C), jnp.float32),
    )(X, W, b.reshape(1, C))


def _tc_pool_wsum(g, val, R):
    D = g.shape[1]
    g3 = g.reshape(R, 3 * D)
    v3 = val.reshape(R, 3)
    BLK = min(1024, _cdiv(R, 8) * 8)
    grid = _cdiv(R, BLK)

    def body(g_ref, v_ref, o_ref):
        gg = g_ref[...]
        vv = v_ref[...]
        o_ref[...] = (gg[:, 0:D] * vv[:, 0:1]
                      + gg[:, D:2 * D] * vv[:, 1:2]
                      + gg[:, 2 * D:3 * D] * vv[:, 2:3])

    return pl.pallas_call(
        body,
        grid=(grid,),
        in_specs=[
            pl.BlockSpec((BLK, 3 * D), lambda i: (i, 0)),
            pl.BlockSpec((BLK, 3), lambda i: (i, 0)),
        ],
        out_specs=pl.BlockSpec((BLK, D), lambda i: (i, 0)),
        out_shape=jax.ShapeDtypeStruct((R, D), jnp.float32),
    )(g3, v3)


def _tc_latent(H, Wmu, bmu, Wdl, bdl):
    bs, F = H.shape
    L = Wmu.shape[0]

    def body(h_ref, wmu_ref, bmu_ref, wdl_ref, bdl_ref, o_ref):
        pre = lax.dot_general(h_ref[...], wmu_ref[...],
                              (((1,), (1,)), ((), ())),
                              preferred_element_type=jnp.float32)
        mu = 1.0 / (1.0 + jnp.exp(-(pre + bmu_ref[...])))
        z = lax.dot_general(mu, wdl_ref[...],
                            (((1,), (1,)), ((), ())),
                            preferred_element_type=jnp.float32)
        o_ref[...] = z + bdl_ref[...]

    return pl.pallas_call(
        body,
        out_shape=jax.ShapeDtypeStruct((bs, F), jnp.float32),
    )(H, Wmu, bmu.reshape(1, L), Wdl, bdl.reshape(1, F))


def _conv_w(W, c_in, cp, bs):
    c_out = W.shape[0]
    Wk = W.reshape(c_out, 9, c_in).transpose(1, 2, 0)
    if cp != c_in:
        Wk = jnp.pad(Wk, ((0, 0), (0, cp - c_in), (0, 0)))
    eye = jnp.eye(bs, dtype=W.dtype)
    W2 = Wk[:, None, :, None, :] * eye[None, :, None, :, None]
    return W2.reshape(9 * bs * cp, bs * c_out)


def kernel(x, idx0, idx1, idx2, idx3, drow0, dcol0, dval0, drow1, dcol1, dval1, drow2, dcol2, dval2, drow3, dcol3, dval3, urow0, ucol0, uval0, urow1, ucol1, uval1, urow2, ucol2, uval2, urow3, ucol3, uval3, We0, be0, We1, be1, We2, be2, We3, be3, Wmu, bmu, Wdl, bdl, Wd1, bd1, Wd2, bd2, Wd3, bd3, Wd4, bd4, Wo, bo):
    bs, n0, _ = x.shape
    sizes = [idx0.shape[0], idx1.shape[0], idx2.shape[0], idx3.shape[0],
             drow3.shape[0] // 3]

    h = jnp.pad(x.transpose(1, 0, 2), ((0, 0), (0, 0), (0, 1))).reshape(n0, bs * 4)
    c_in, cp = 3, 4
    enc = [(idx0, We0, be0, dcol0, dval0, sizes[1]),
           (idx1, We1, be1, dcol1, dval1, sizes[2]),
           (idx2, We2, be2, dcol2, dval2, sizes[3]),
           (idx3, We3, be3, dcol3, dval3, sizes[4])]
    for idx, We, be, dcol, dval, R in enc:
        n = idx.shape[0]
        c_out = We.shape[0]
        g = _sc_gather(h, idx.reshape(-1))
        X = g.reshape(n, 9 * bs * cp)
        bb = jnp.tile(be, bs)
        h2 = _tc_linear(X, _conv_w(We, c_in, cp, bs), bb, "elu")
        pg = _sc_gather(h2, dcol)
        h = _tc_pool_wsum(pg, dval, R)
        c_in = cp = c_out

    R4 = sizes[4]
    H = h.reshape(R4, bs, 32).transpose(1, 0, 2).reshape(bs, R4 * 32)
    z2 = _tc_latent(H, Wmu, bmu, Wdl, bdl)
    z = z2.reshape(bs, R4, 32).transpose(1, 0, 2).reshape(R4, bs * 32)
    c = 32

    dec = [(ucol3, uval3, sizes[3], idx3, Wd1, bd1),
           (ucol2, uval2, sizes[2], idx2, Wd2, bd2),
           (ucol1, uval1, sizes[1], idx1, Wd3, bd3),
           (ucol0, uval0, sizes[0], idx0, Wd4, bd4)]
    for ucol, uval, R, idx, Wd, bd in dec:
        pg = _sc_gather(z, ucol)
        p = _tc_pool_wsum(pg, uval, R)
        g = _sc_gather(p, idx.reshape(-1))
        X = g.reshape(R, 9 * bs * c)
        z = _tc_linear(X, _conv_w(Wd, c, c, bs), jnp.tile(bd, bs), "elu")
        c = Wd.shape[0]

    g = _sc_gather(z, idx0.reshape(-1))
    out = _tc_linear(g.reshape(n0, 9 * bs * c), _conv_w(Wo, c, c, bs),
                     jnp.tile(bo, bs), None)
    return out.reshape(n0, bs, 3).transpose(1, 0, 2)


# speedup vs baseline: 6.9358x; 1.6182x over previous
import functools

import jax
import jax.numpy as jnp
from jax import lax
from jax.experimental import pallas as pl
from jax.experimental.pallas import tpu as pltpu
from jax.experimental.pallas import tpu_sc as plsc

NC = 2
NS = 16
NW = NC * NS


def _cdiv(a, b):
    return -(-a // b)


@functools.lru_cache(maxsize=None)
def _sc_gather_fn(N, D, M_pad, ch, iters, per_w):
    mesh = plsc.VectorSubcoreMesh(core_axis_name="c", subcore_axis_name="s")

    @functools.partial(
        pl.kernel,
        out_type=jax.ShapeDtypeStruct((M_pad, D), jnp.float32),
        mesh=mesh,
        scratch_types=[
            pltpu.VMEM((ch,), jnp.int32),
            pltpu.VMEM((ch,), jnp.int32),
            pltpu.VMEM((ch, D), jnp.float32),
            pltpu.VMEM((ch, D), jnp.float32),
            pltpu.SemaphoreType.DMA,
            pltpu.SemaphoreType.DMA,
            pltpu.SemaphoreType.DMA,
            pltpu.SemaphoreType.DMA,
        ],
        compiler_params=pltpu.CompilerParams(use_tc_tiling_on_sc=False),
    )
    def k(table_hbm, idx_hbm, out_hbm, i0, i1, r0, r1, sg0, sg1, ss0, ss1):
        wid = lax.axis_index("s") * NC + lax.axis_index("c")
        base = wid * per_w
        idx_v = [i0, i1]
        rows_v = [r0, r1]
        sg = [sg0, sg1]
        ss = [ss0, ss1]

        pltpu.sync_copy(idx_hbm.at[pl.ds(base, ch)], idx_v[0])
        gh = [None, None]
        sh = [None, None]
        gh[0] = pltpu.async_copy(table_hbm.at[idx_v[0]], rows_v[0], sg[0])
        for g in range(iters):
            b = g % 2
            nb = (g + 1) % 2
            if g + 1 < iters:
                if sh[nb] is not None:
                    sh[nb].wait()
                pltpu.sync_copy(idx_hbm.at[pl.ds(base + (g + 1) * ch, ch)],
                                idx_v[nb])
                gh[nb] = pltpu.async_copy(table_hbm.at[idx_v[nb]],
                                          rows_v[nb], sg[nb])
            gh[b].wait()
            sh[b] = pltpu.async_copy(rows_v[b],
                                     out_hbm.at[pl.ds(base + g * ch, ch)],
                                     ss[b])
        for b in range(2):
            if sh[b] is not None:
                sh[b].wait()

    return k


def _sc_gather(table, idx):
    N, D = table.shape
    M = idx.shape[0]
    ch_cap = max(8, (480_000 // (8 * D + 8)) // 8 * 8)
    per_w = _cdiv(M, NW)
    ch = min(ch_cap, _cdiv(per_w, 8) * 8)
    iters = _cdiv(per_w, ch)
    per_w = iters * ch
    M_pad = per_w * NW
    if M_pad != M:
        idx = jnp.pad(idx, (0, M_pad - M))
    out = _sc_gather_fn(N, D, M_pad, ch, iters, per_w)(table, idx)
    if M_pad != M:
        out = out[:M]
    return out


def _tc_linear(X, W, b, act):
    Nr, K = X.shape
    C = W.shape[1]
    BLK = min(1024, _cdiv(Nr, 8) * 8)
    grid = _cdiv(Nr, BLK)

    def body(x_ref, w_ref, b_ref, o_ref):
        acc = jnp.dot(x_ref[...], w_ref[...],
                      preferred_element_type=jnp.float32) + b_ref[...]
        if act == "elu":
            acc = jnp.where(acc > 0, acc, jnp.exp(jnp.minimum(acc, 0.0)) - 1.0)
        o_ref[...] = acc

    return pl.pallas_call(
        body,
        grid=(grid,),
        in_specs=[
            pl.BlockSpec((BLK, K), lambda i: (i, 0)),
            pl.BlockSpec((K, C), lambda i: (0, 0)),
            pl.BlockSpec((1, C), lambda i: (0, 0)),
        ],
        out_specs=pl.BlockSpec((BLK, C), lambda i: (i, 0)),
        out_shape=jax.ShapeDtypeStruct((Nr, C), jnp.float32),
    )(X, W, b.reshape(1, C))


def _tc_pool_wsum(g, val, R):
    D = g.shape[1]
    g3 = g.reshape(R, 3 * D)
    v3 = val.reshape(R, 3)
    BLK = min(1024, _cdiv(R, 8) * 8)
    grid = _cdiv(R, BLK)

    def body(g_ref, v_ref, o_ref):
        gg = g_ref[...]
        vv = v_ref[...]
        o_ref[...] = (gg[:, 0:D] * vv[:, 0:1]
                      + gg[:, D:2 * D] * vv[:, 1:2]
                      + gg[:, 2 * D:3 * D] * vv[:, 2:3])

    return pl.pallas_call(
        body,
        grid=(grid,),
        in_specs=[
            pl.BlockSpec((BLK, 3 * D), lambda i: (i, 0)),
            pl.BlockSpec((BLK, 3), lambda i: (i, 0)),
        ],
        out_specs=pl.BlockSpec((BLK, D), lambda i: (i, 0)),
        out_shape=jax.ShapeDtypeStruct((R, D), jnp.float32),
    )(g3, v3)


def _tc_latent(H, Wmu, bmu, Wdl, bdl):
    bs, F = H.shape
    L = Wmu.shape[0]

    def body(h_ref, wmu_ref, bmu_ref, wdl_ref, bdl_ref, o_ref):
        pre = lax.dot_general(h_ref[...], wmu_ref[...],
                              (((1,), (1,)), ((), ())),
                              preferred_element_type=jnp.float32)
        mu = 1.0 / (1.0 + jnp.exp(-(pre + bmu_ref[...])))
        z = lax.dot_general(mu, wdl_ref[...],
                            (((1,), (1,)), ((), ())),
                            preferred_element_type=jnp.float32)
        o_ref[...] = z + bdl_ref[...]

    return pl.pallas_call(
        body,
        out_shape=jax.ShapeDtypeStruct((bs, F), jnp.float32),
    )(H, Wmu, bmu.reshape(1, L), Wdl, bdl.reshape(1, F))


def _conv_w(W, c_in, cp, bs):
    c_out = W.shape[0]
    Wk = W.reshape(c_out, 9, c_in).transpose(1, 2, 0)
    if cp != c_in:
        Wk = jnp.pad(Wk, ((0, 0), (0, cp - c_in), (0, 0)))
    eye = jnp.eye(bs, dtype=W.dtype)
    W2 = Wk[:, None, :, None, :] * eye[None, :, None, :, None]
    return W2.reshape(9 * bs * cp, bs * c_out)


def kernel(x, idx0, idx1, idx2, idx3, drow0, dcol0, dval0, drow1, dcol1, dval1, drow2, dcol2, dval2, drow3, dcol3, dval3, urow0, ucol0, uval0, urow1, ucol1, uval1, urow2, ucol2, uval2, urow3, ucol3, uval3, We0, be0, We1, be1, We2, be2, We3, be3, Wmu, bmu, Wdl, bdl, Wd1, bd1, Wd2, bd2, Wd3, bd3, Wd4, bd4, Wo, bo):
    bs, n0, _ = x.shape
    sizes = [idx0.shape[0], idx1.shape[0], idx2.shape[0], idx3.shape[0],
             drow3.shape[0] // 3]

    h = jnp.pad(x.transpose(1, 0, 2), ((0, 0), (0, 0), (0, 1))).reshape(n0, bs * 4)
    c_in, cp = 3, 4
    enc = [(idx0, We0, be0, dcol0, dval0, sizes[1]),
           (idx1, We1, be1, dcol1, dval1, sizes[2]),
           (idx2, We2, be2, dcol2, dval2, sizes[3]),
           (idx3, We3, be3, dcol3, dval3, sizes[4])]
    for idx, We, be, dcol, dval, R in enc:
        n = idx.shape[0]
        c_out = We.shape[0]
        g = _sc_gather(h, idx.reshape(-1))
        X = g.reshape(n, 9 * bs * cp)
        bb = jnp.tile(be, bs)
        h2 = _tc_linear(X, _conv_w(We, c_in, cp, bs), bb, "elu")
        pg = _sc_gather(h2, dcol)
        h = _tc_pool_wsum(pg, dval, R)
        c_in = cp = c_out

    R4 = sizes[4]
    H = h.reshape(R4, bs, 32).transpose(1, 0, 2).reshape(bs, R4 * 32)
    z2 = _tc_latent(H, Wmu, bmu, Wdl, bdl)
    z = z2.reshape(bs, R4, 32).transpose(1, 0, 2).reshape(R4, bs * 32)
    c = 32

    dec = [(ucol3, uval3, sizes[3], idx3, Wd1, bd1),
           (ucol2, uval2, sizes[2], idx2, Wd2, bd2),
           (ucol1, uval1, sizes[1], idx1, Wd3, bd3),
           (ucol0, uval0, sizes[0], idx0, Wd4, bd4)]
    for ucol, uval, R, idx, Wd, bd in dec:
        pg = _sc_gather(z, ucol)
        p = _tc_pool_wsum(pg, uval, R)
        g = _sc_gather(p, idx.reshape(-1))
        X = g.reshape(R, 9 * bs * c)
        z = _tc_linear(X, _conv_w(Wd, c, c, bs), jnp.tile(bd, bs), "elu")
        c = Wd.shape[0]

    g = _sc_gather(z, idx0.reshape(-1))
    out = _tc_linear(g.reshape(n0, 9 * bs * c), _conv_w(Wo, c, c, bs),
                     jnp.tile(bo, bs), None)
    return out.reshape(n0, bs, 3).transpose(1, 0, 2)


# speedup vs baseline: 9.6747x; 1.3949x over previous
import functools

import jax
import jax.numpy as jnp
from jax import lax
from jax.experimental import pallas as pl
from jax.experimental.pallas import tpu as pltpu
from jax.experimental.pallas import tpu_sc as plsc

NC = 2
NS = 16
NW = NC * NS


def _cdiv(a, b):
    return -(-a // b)


@functools.lru_cache(maxsize=None)
def _sc_gather_fn(N, D, M_pad, ch, iters, per_w):
    mesh = plsc.VectorSubcoreMesh(core_axis_name="c", subcore_axis_name="s")

    @functools.partial(
        pl.kernel,
        out_type=jax.ShapeDtypeStruct((M_pad, D), jnp.float32),
        mesh=mesh,
        scratch_types=(
            [pltpu.VMEM((ch,), jnp.int32) for _ in range(_NB)]
            + [pltpu.VMEM((ch, D), jnp.float32) for _ in range(_NB)]
            + [pltpu.SemaphoreType.DMA for _ in range(2 * _NB)]
        ),
        compiler_params=pltpu.CompilerParams(use_tc_tiling_on_sc=False),
    )
    def k(table_hbm, idx_hbm, out_hbm, *scr):
        idx_v = scr[:_NB]
        rows_v = scr[_NB:2 * _NB]
        sg = scr[2 * _NB:3 * _NB]
        ss = scr[3 * _NB:4 * _NB]
        wid = lax.axis_index("s") * NC + lax.axis_index("c")
        base = wid * per_w

        gh = [None] * _NB
        sh = [None] * _NB
        depth = min(_NB - 1, iters)
        for g in range(depth):
            pltpu.sync_copy(idx_hbm.at[pl.ds(base + g * ch, ch)], idx_v[g])
            gh[g] = pltpu.async_copy(table_hbm.at[idx_v[g]], rows_v[g], sg[g])
        for g in range(iters):
            b = g % _NB
            if g + depth < iters:
                nb = (g + depth) % _NB
                if sh[nb] is not None:
                    sh[nb].wait()
                pltpu.sync_copy(idx_hbm.at[pl.ds(base + (g + depth) * ch, ch)],
                                idx_v[nb])
                gh[nb] = pltpu.async_copy(table_hbm.at[idx_v[nb]],
                                          rows_v[nb], sg[nb])
            gh[b].wait()
            sh[b] = pltpu.async_copy(rows_v[b],
                                     out_hbm.at[pl.ds(base + g * ch, ch)],
                                     ss[b])
        for b in range(_NB):
            if sh[b] is not None:
                sh[b].wait()

    return k


_NB = 4


def _sc_gather(table, idx):
    N, D = table.shape
    M = idx.shape[0]
    ch_cap = max(8, (480_000 // (_NB * (4 * D + 4))) // 8 * 8)
    per_w = _cdiv(M, NW)
    ch = min(ch_cap, _cdiv(per_w, 8) * 8)
    iters = _cdiv(per_w, ch)
    per_w = iters * ch
    M_pad = per_w * NW
    if M_pad != M:
        idx = jnp.pad(idx, (0, M_pad - M))
    out = _sc_gather_fn(N, D, M_pad, ch, iters, per_w)(table, idx)
    if M_pad != M:
        out = out[:M]
    return out


def _tc_linear(X, W, b, act):
    Nr, K = X.shape
    C = W.shape[1]
    BLK = min(1024, _cdiv(Nr, 8) * 8)
    grid = _cdiv(Nr, BLK)

    def body(x_ref, w_ref, b_ref, o_ref):
        acc = jnp.dot(x_ref[...], w_ref[...],
                      preferred_element_type=jnp.float32) + b_ref[...]
        if act == "elu":
            acc = jnp.where(acc > 0, acc, jnp.exp(jnp.minimum(acc, 0.0)) - 1.0)
        o_ref[...] = acc

    return pl.pallas_call(
        body,
        grid=(grid,),
        in_specs=[
            pl.BlockSpec((BLK, K), lambda i: (i, 0)),
            pl.BlockSpec((K, C), lambda i: (0, 0)),
            pl.BlockSpec((1, C), lambda i: (0, 0)),
        ],
        out_specs=pl.BlockSpec((BLK, C), lambda i: (i, 0)),
        out_shape=jax.ShapeDtypeStruct((Nr, C), jnp.float32),
    )(X, W, b.reshape(1, C))


def _tc_pool_wsum(g, val, R):
    D = g.shape[1]
    g3 = g.reshape(R, 3 * D)
    v3 = val.reshape(R, 3)
    BLK = min(1024, _cdiv(R, 8) * 8)
    grid = _cdiv(R, BLK)

    def body(g_ref, v_ref, o_ref):
        gg = g_ref[...]
        vv = v_ref[...]
        o_ref[...] = (gg[:, 0:D] * vv[:, 0:1]
                      + gg[:, D:2 * D] * vv[:, 1:2]
                      + gg[:, 2 * D:3 * D] * vv[:, 2:3])

    return pl.pallas_call(
        body,
        grid=(grid,),
        in_specs=[
            pl.BlockSpec((BLK, 3 * D), lambda i: (i, 0)),
            pl.BlockSpec((BLK, 3), lambda i: (i, 0)),
        ],
        out_specs=pl.BlockSpec((BLK, D), lambda i: (i, 0)),
        out_shape=jax.ShapeDtypeStruct((R, D), jnp.float32),
    )(g3, v3)


def _tc_latent(H, Wmu, bmu, Wdl, bdl):
    bs, F = H.shape
    L = Wmu.shape[0]

    def body(h_ref, wmu_ref, bmu_ref, wdl_ref, bdl_ref, o_ref):
        pre = lax.dot_general(h_ref[...], wmu_ref[...],
                              (((1,), (1,)), ((), ())),
                              preferred_element_type=jnp.float32)
        mu = 1.0 / (1.0 + jnp.exp(-(pre + bmu_ref[...])))
        z = lax.dot_general(mu, wdl_ref[...],
                            (((1,), (1,)), ((), ())),
                            preferred_element_type=jnp.float32)
        o_ref[...] = z + bdl_ref[...]

    return pl.pallas_call(
        body,
        out_shape=jax.ShapeDtypeStruct((bs, F), jnp.float32),
    )(H, Wmu, bmu.reshape(1, L), Wdl, bdl.reshape(1, F))


def _conv_w(W, c_in, cp, bs):
    c_out = W.shape[0]
    Wk = W.reshape(c_out, 9, c_in).transpose(1, 2, 0)
    if cp != c_in:
        Wk = jnp.pad(Wk, ((0, 0), (0, cp - c_in), (0, 0)))
    eye = jnp.eye(bs, dtype=W.dtype)
    W2 = Wk[:, None, :, None, :] * eye[None, :, None, :, None]
    return W2.reshape(9 * bs * cp, bs * c_out)


def kernel(x, idx0, idx1, idx2, idx3, drow0, dcol0, dval0, drow1, dcol1, dval1, drow2, dcol2, dval2, drow3, dcol3, dval3, urow0, ucol0, uval0, urow1, ucol1, uval1, urow2, ucol2, uval2, urow3, ucol3, uval3, We0, be0, We1, be1, We2, be2, We3, be3, Wmu, bmu, Wdl, bdl, Wd1, bd1, Wd2, bd2, Wd3, bd3, Wd4, bd4, Wo, bo):
    bs, n0, _ = x.shape
    sizes = [idx0.shape[0], idx1.shape[0], idx2.shape[0], idx3.shape[0],
             drow3.shape[0] // 3]

    h = jnp.pad(x.transpose(1, 0, 2), ((0, 0), (0, 0), (0, 1))).reshape(n0, bs * 4)
    c_in, cp = 3, 4
    enc = [(idx0, We0, be0, dcol0, dval0, sizes[1]),
           (idx1, We1, be1, dcol1, dval1, sizes[2]),
           (idx2, We2, be2, dcol2, dval2, sizes[3]),
           (idx3, We3, be3, dcol3, dval3, sizes[4])]
    for idx, We, be, dcol, dval, R in enc:
        n = idx.shape[0]
        c_out = We.shape[0]
        g = _sc_gather(h, idx.reshape(-1))
        X = g.reshape(n, 9 * bs * cp)
        bb = jnp.tile(be, bs)
        h2 = _tc_linear(X, _conv_w(We, c_in, cp, bs), bb, "elu")
        pg = _sc_gather(h2, dcol)
        h = _tc_pool_wsum(pg, dval, R)
        c_in = cp = c_out

    R4 = sizes[4]
    H = h.reshape(R4, bs, 32).transpose(1, 0, 2).reshape(bs, R4 * 32)
    z2 = _tc_latent(H, Wmu, bmu, Wdl, bdl)
    z = z2.reshape(bs, R4, 32).transpose(1, 0, 2).reshape(R4, bs * 32)
    c = 32

    dec = [(ucol3, uval3, sizes[3], idx3, Wd1, bd1),
           (ucol2, uval2, sizes[2], idx2, Wd2, bd2),
           (ucol1, uval1, sizes[1], idx1, Wd3, bd3),
           (ucol0, uval0, sizes[0], idx0, Wd4, bd4)]
    for ucol, uval, R, idx, Wd, bd in dec:
        pg = _sc_gather(z, ucol)
        p = _tc_pool_wsum(pg, uval, R)
        g = _sc_gather(p, idx.reshape(-1))
        X = g.reshape(R, 9 * bs * c)
        z = _tc_linear(X, _conv_w(Wd, c, c, bs), jnp.tile(bd, bs), "elu")
        c = Wd.shape[0]

    g = _sc_gather(z, idx0.reshape(-1))
    out = _tc_linear(g.reshape(n0, 9 * bs * c), _conv_w(Wo, c, c, bs),
                     jnp.tile(bo, bs), None)
    return out.reshape(n0, bs, 3).transpose(1, 0, 2)


# speedup vs baseline: 9.8912x; 1.0224x over previous
import functools

import jax
import jax.numpy as jnp
from jax import lax
from jax.experimental import pallas as pl
from jax.experimental.pallas import tpu as pltpu
from jax.experimental.pallas import tpu_sc as plsc

NC = 2
NS = 16
NW = NC * NS


def _cdiv(a, b):
    return -(-a // b)


@functools.lru_cache(maxsize=None)
def _sc_gather_fn(N, D, M_pad, ch, iters, per_w):
    mesh = plsc.VectorSubcoreMesh(core_axis_name="c", subcore_axis_name="s")

    @functools.partial(
        pl.kernel,
        out_type=jax.ShapeDtypeStruct((M_pad, D), jnp.float32),
        mesh=mesh,
        scratch_types=(
            [pltpu.VMEM((ch,), jnp.int32) for _ in range(_NB)]
            + [pltpu.VMEM((ch, D), jnp.float32) for _ in range(_NB)]
            + [pltpu.SemaphoreType.DMA for _ in range(2 * _NB)]
        ),
        compiler_params=pltpu.CompilerParams(use_tc_tiling_on_sc=False),
    )
    def k(table_hbm, idx_hbm, out_hbm, *scr):
        idx_v = scr[:_NB]
        rows_v = scr[_NB:2 * _NB]
        sg = scr[2 * _NB:3 * _NB]
        ss = scr[3 * _NB:4 * _NB]
        wid = lax.axis_index("s") * NC + lax.axis_index("c")
        base = wid * per_w

        gh = [None] * _NB
        sh = [None] * _NB
        depth = min(_NB - 1, iters)
        for g in range(depth):
            pltpu.sync_copy(idx_hbm.at[pl.ds(base + g * ch, ch)], idx_v[g])
            gh[g] = pltpu.async_copy(table_hbm.at[idx_v[g]], rows_v[g], sg[g])
        for g in range(iters):
            b = g % _NB
            if g + depth < iters:
                nb = (g + depth) % _NB
                if sh[nb] is not None:
                    sh[nb].wait()
                pltpu.sync_copy(idx_hbm.at[pl.ds(base + (g + depth) * ch, ch)],
                                idx_v[nb])
                gh[nb] = pltpu.async_copy(table_hbm.at[idx_v[nb]],
                                          rows_v[nb], sg[nb])
            gh[b].wait()
            sh[b] = pltpu.async_copy(rows_v[b],
                                     out_hbm.at[pl.ds(base + g * ch, ch)],
                                     ss[b])
        for b in range(_NB):
            if sh[b] is not None:
                sh[b].wait()

    return k


_NB = 6


def _sc_gather(table, idx):
    N, D = table.shape
    M = idx.shape[0]
    ch_cap = max(8, (480_000 // (_NB * (4 * D + 4))) // 8 * 8)
    per_w = _cdiv(M, NW)
    ch = min(ch_cap, _cdiv(per_w, 8) * 8)
    iters = _cdiv(per_w, ch)
    per_w = iters * ch
    M_pad = per_w * NW
    if M_pad != M:
        idx = jnp.pad(idx, (0, M_pad - M))
    out = _sc_gather_fn(N, D, M_pad, ch, iters, per_w)(table, idx)
    if M_pad != M:
        out = out[:M]
    return out


def _tc_linear(X, W, b, act):
    Nr, K = X.shape
    C = W.shape[1]
    BLK = min(1024, _cdiv(Nr, 8) * 8)
    grid = _cdiv(Nr, BLK)

    def body(x_ref, w_ref, b_ref, o_ref):
        acc = jnp.dot(x_ref[...], w_ref[...],
                      preferred_element_type=jnp.float32) + b_ref[...]
        if act == "elu":
            acc = jnp.where(acc > 0, acc, jnp.exp(jnp.minimum(acc, 0.0)) - 1.0)
        o_ref[...] = acc

    return pl.pallas_call(
        body,
        grid=(grid,),
        in_specs=[
            pl.BlockSpec((BLK, K), lambda i: (i, 0)),
            pl.BlockSpec((K, C), lambda i: (0, 0)),
            pl.BlockSpec((1, C), lambda i: (0, 0)),
        ],
        out_specs=pl.BlockSpec((BLK, C), lambda i: (i, 0)),
        out_shape=jax.ShapeDtypeStruct((Nr, C), jnp.float32),
    )(X, W, b.reshape(1, C))


def _tc_pool_wsum(g, val, R):
    D = g.shape[1]
    g3 = g.reshape(R, 3 * D)
    v3 = val.reshape(R, 3)
    BLK = min(1024, _cdiv(R, 8) * 8)
    grid = _cdiv(R, BLK)

    def body(g_ref, v_ref, o_ref):
        gg = g_ref[...]
        vv = v_ref[...]
        o_ref[...] = (gg[:, 0:D] * vv[:, 0:1]
                      + gg[:, D:2 * D] * vv[:, 1:2]
                      + gg[:, 2 * D:3 * D] * vv[:, 2:3])

    return pl.pallas_call(
        body,
        grid=(grid,),
        in_specs=[
            pl.BlockSpec((BLK, 3 * D), lambda i: (i, 0)),
            pl.BlockSpec((BLK, 3), lambda i: (i, 0)),
        ],
        out_specs=pl.BlockSpec((BLK, D), lambda i: (i, 0)),
        out_shape=jax.ShapeDtypeStruct((R, D), jnp.float32),
    )(g3, v3)


def _tc_latent(H, Wmu, bmu, Wdl, bdl):
    bs, F = H.shape
    L = Wmu.shape[0]

    def body(h_ref, wmu_ref, bmu_ref, wdl_ref, bdl_ref, o_ref):
        pre = lax.dot_general(h_ref[...], wmu_ref[...],
                              (((1,), (1,)), ((), ())),
                              preferred_element_type=jnp.float32)
        mu = 1.0 / (1.0 + jnp.exp(-(pre + bmu_ref[...])))
        z = lax.dot_general(mu, wdl_ref[...],
                            (((1,), (1,)), ((), ())),
                            preferred_element_type=jnp.float32)
        o_ref[...] = z + bdl_ref[...]

    return pl.pallas_call(
        body,
        out_shape=jax.ShapeDtypeStruct((bs, F), jnp.float32),
    )(H, Wmu, bmu.reshape(1, L), Wdl, bdl.reshape(1, F))


def _conv_w(W, c_in, cp, bs):
    c_out = W.shape[0]
    Wk = W.reshape(c_out, 9, c_in).transpose(1, 2, 0)
    if cp != c_in:
        Wk = jnp.pad(Wk, ((0, 0), (0, cp - c_in), (0, 0)))
    eye = jnp.eye(bs, dtype=W.dtype)
    W2 = Wk[:, None, :, None, :] * eye[None, :, None, :, None]
    return W2.reshape(9 * bs * cp, bs * c_out)


def kernel(x, idx0, idx1, idx2, idx3, drow0, dcol0, dval0, drow1, dcol1, dval1, drow2, dcol2, dval2, drow3, dcol3, dval3, urow0, ucol0, uval0, urow1, ucol1, uval1, urow2, ucol2, uval2, urow3, ucol3, uval3, We0, be0, We1, be1, We2, be2, We3, be3, Wmu, bmu, Wdl, bdl, Wd1, bd1, Wd2, bd2, Wd3, bd3, Wd4, bd4, Wo, bo):
    bs, n0, _ = x.shape
    sizes = [idx0.shape[0], idx1.shape[0], idx2.shape[0], idx3.shape[0],
             drow3.shape[0] // 3]

    h = jnp.pad(x.transpose(1, 0, 2), ((0, 0), (0, 0), (0, 1))).reshape(n0, bs * 4)
    c_in, cp = 3, 4
    enc = [(idx0, We0, be0, dcol0, dval0, sizes[1]),
           (idx1, We1, be1, dcol1, dval1, sizes[2]),
           (idx2, We2, be2, dcol2, dval2, sizes[3]),
           (idx3, We3, be3, dcol3, dval3, sizes[4])]
    for idx, We, be, dcol, dval, R in enc:
        n = idx.shape[0]
        c_out = We.shape[0]
        g = _sc_gather(h, idx.reshape(-1))
        X = g.reshape(n, 9 * bs * cp)
        bb = jnp.tile(be, bs)
        h2 = _tc_linear(X, _conv_w(We, c_in, cp, bs), bb, "elu")
        pg = _sc_gather(h2, dcol)
        h = _tc_pool_wsum(pg, dval, R)
        c_in = cp = c_out

    R4 = sizes[4]
    H = h.reshape(R4, bs, 32).transpose(1, 0, 2).reshape(bs, R4 * 32)
    z2 = _tc_latent(H, Wmu, bmu, Wdl, bdl)
    z = z2.reshape(bs, R4, 32).transpose(1, 0, 2).reshape(R4, bs * 32)
    c = 32

    dec = [(ucol3, uval3, sizes[3], idx3, Wd1, bd1),
           (ucol2, uval2, sizes[2], idx2, Wd2, bd2),
           (ucol1, uval1, sizes[1], idx1, Wd3, bd3),
           (ucol0, uval0, sizes[0], idx0, Wd4, bd4)]
    for ucol, uval, R, idx, Wd, bd in dec:
        pg = _sc_gather(z, ucol)
        p = _tc_pool_wsum(pg, uval, R)
        g = _sc_gather(p, idx.reshape(-1))
        X = g.reshape(R, 9 * bs * c)
        z = _tc_linear(X, _conv_w(Wd, c, c, bs), jnp.tile(bd, bs), "elu")
        c = Wd.shape[0]

    g = _sc_gather(z, idx0.reshape(-1))
    out = _tc_linear(g.reshape(n0, 9 * bs * c), _conv_w(Wo, c, c, bs),
                     jnp.tile(bo, bs), None)
    return out.reshape(n0, bs, 3).transpose(1, 0, 2)
